# Initial kernel scaffold; baseline (speedup 1.0000x reference)
import jax
import jax.numpy as jnp
from jax import lax
from jax.experimental import pallas as pl
from jax.experimental.pallas import tpu as pltpu
from jax.experimental.pallas import tpu_sc as plsc

NC, NS, LANES = 2, 16, 16
NW = NC * NS
B = 4096 * 50
D = 64
PER_W = B // NW
CH = 128
NCH = PER_W // CH
R = 5
MAP_AHEAD = 3


def _body(text_hbm, tw_hbm, fw_hbm, tmap_hbm, fmap_hbm, out_hbm,
          tok_v, tidx_v, fidx_v, rows_a, rows_b, sem_row, sem_out, sem_map):
    wid = lax.axis_index("s") * NC + lax.axis_index("c")
    tok_row0 = wid * NCH

    pltpu.sync_copy(text_hbm.at[pl.ds(tok_row0, NCH)], tok_v)

    def map_copies(c, s):
        return (
            pltpu.make_async_copy(tmap_hbm.at[tok_v.at[c]], tidx_v.at[c],
                                  sem_map.at[s]),
            pltpu.make_async_copy(fmap_hbm.at[tok_v.at[c]], fidx_v.at[c],
                                  sem_map.at[s]),
        )

    def row_copies(c, s):
        return (
            pltpu.make_async_copy(tw_hbm.at[tidx_v.at[c]], rows_a[s],
                                  sem_row.at[s]),
            pltpu.make_async_copy(fw_hbm.at[fidx_v.at[c]], rows_b[s],
                                  sem_row.at[s]),
        )

    def out_copy(c, s):
        return pltpu.make_async_copy(
            rows_a[s], out_hbm.at[pl.ds(wid * PER_W + c * CH, CH)],
            sem_out.at[s])

    def issue(copies):
        for cp in copies:
            cp.start()

    def drain(copies):
        for cp in copies:
            cp.wait()

    def compute(s):
        @pl.loop(0, CH)
        def _(r):
            for cc in range(D // LANES):
                sl = pl.ds(cc * LANES, LANES)
                rows_a[s][r, sl] = rows_a[s][r, sl] + rows_b[s][r, sl]

    for c in range(MAP_AHEAD):
        issue(map_copies(c, c % R))
    drain(map_copies(0, 0))
    issue(row_copies(0, 0))

    @pl.loop(0, NCH, step=R)
    def _(i0):
        for b in range(R):
            i = i0 + b
            drain(row_copies(i, b))
            compute(b)
            out_copy(i, b).start()

            j = i + 1
            bj = (b + 1) % R

            @pl.when(j < NCH)
            def _():
                drain(map_copies(j, bj))

                @pl.when(j >= R)
                def _():
                    out_copy(j - R, bj).wait()

                issue(row_copies(j, bj))

            m = i + MAP_AHEAD
            bm = (b + MAP_AHEAD) % R

            @pl.when(m < NCH)
            def _():
                issue(map_copies(m, bm))

    for b in range(R):
        out_copy(NCH - R + b, b).wait()


_run = pl.kernel(
    _body,
    out_type=jax.ShapeDtypeStruct((B, D), jnp.float32),
    mesh=plsc.VectorSubcoreMesh(core_axis_name="c", subcore_axis_name="s"),
    scratch_types=[
        pltpu.VMEM((NCH, CH), jnp.int32),
        pltpu.VMEM((NCH, CH), jnp.int32),
        pltpu.VMEM((NCH, CH), jnp.int32),
        [pltpu.VMEM((CH, D), jnp.float32) for _ in range(R)],
        [pltpu.VMEM((CH, D), jnp.float32) for _ in range(R)],
        pltpu.SemaphoreType.DMA((R,)),
        pltpu.SemaphoreType.DMA((R,)),
        pltpu.SemaphoreType.DMA((R,)),
    ],
)


@jax.jit
def kernel(text_input, trainable_weight, frozen_weight, trainable_map,
           frozen_map):
    text2d = text_input.reshape(B // CH, CH)
    out = _run(text2d, trainable_weight, frozen_weight, trainable_map,
               frozen_map)
    return out.reshape(text_input.shape[0], text_input.shape[1], D)


# speedup vs baseline: 1.3898x; 1.3898x over previous
import jax
import jax.numpy as jnp
from jax import lax
from jax.experimental import pallas as pl
from jax.experimental.pallas import tpu as pltpu
from jax.experimental.pallas import tpu_sc as plsc

NC, NS, LANES = 2, 16, 16
NW = NC * NS
B = 4096 * 50
D = 64
PER_W = B // NW
CH = 128
NCH = PER_W // CH
R = 5
MAP_AHEAD = 3


def _body(text_hbm, tw_hbm, fw_hbm, tmap_hbm, fmap_hbm, out_hbm,
          tok_v, tidx_v, fidx_v, rows_a, rows_b, sem_row, sem_out, sem_map):
    wid = lax.axis_index("s") * NC + lax.axis_index("c")

    pltpu.sync_copy(text_hbm.at[wid], tok_v)

    def map_copies(c, s):
        return (
            pltpu.make_async_copy(tmap_hbm.at[tok_v.at[c]], tidx_v.at[c],
                                  sem_map.at[s]),
            pltpu.make_async_copy(fmap_hbm.at[tok_v.at[c]], fidx_v.at[c],
                                  sem_map.at[s]),
        )

    def row_copies(c, s):
        return (
            pltpu.make_async_copy(tw_hbm.at[tidx_v.at[c]], rows_a[s],
                                  sem_row.at[s]),
            pltpu.make_async_copy(fw_hbm.at[fidx_v.at[c]], rows_b[s],
                                  sem_row.at[s]),
        )

    def out_copy(c, s):
        return pltpu.make_async_copy(
            rows_a[s], out_hbm.at[pl.ds(wid * PER_W + c * CH, CH)],
            sem_out.at[s])

    def issue(copies):
        for cp in copies:
            cp.start()

    def drain(copies):
        for cp in copies:
            cp.wait()

    def compute(s):
        @pl.loop(0, CH)
        def _(r):
            for cc in range(D // LANES):
                sl = pl.ds(cc * LANES, LANES)
                rows_a[s][r, sl] = rows_a[s][r, sl] + rows_b[s][r, sl]

    for c in range(MAP_AHEAD):
        issue(map_copies(c, c % R))
    drain(map_copies(0, 0))
    issue(row_copies(0, 0))

    @pl.loop(0, NCH, step=R)
    def _(i0):
        for b in range(R):
            i = i0 + b
            drain(row_copies(i, b))
            compute(b)
            out_copy(i, b).start()

            j = i + 1
            bj = (b + 1) % R

            @pl.when(j < NCH)
            def _():
                drain(map_copies(j, bj))

                @pl.when(j >= R)
                def _():
                    out_copy(j - R, bj).wait()

                issue(row_copies(j, bj))

            m = i + MAP_AHEAD
            bm = (b + MAP_AHEAD) % R

            @pl.when(m < NCH)
            def _():
                issue(map_copies(m, bm))

    for b in range(R):
        out_copy(NCH - R + b, b).wait()


_run = pl.kernel(
    _body,
    out_type=jax.ShapeDtypeStruct((B, D), jnp.float32),
    mesh=plsc.VectorSubcoreMesh(core_axis_name="c", subcore_axis_name="s"),
    compiler_params=pltpu.CompilerParams(use_tc_tiling_on_sc=False),
    scratch_types=[
        pltpu.VMEM((NCH, CH), jnp.int32),
        pltpu.VMEM((NCH, CH), jnp.int32),
        pltpu.VMEM((NCH, CH), jnp.int32),
        [pltpu.VMEM((CH, D), jnp.float32) for _ in range(R)],
        [pltpu.VMEM((CH, D), jnp.float32) for _ in range(R)],
        pltpu.SemaphoreType.DMA((R,)),
        pltpu.SemaphoreType.DMA((R,)),
        pltpu.SemaphoreType.DMA((R,)),
    ],
)


@jax.jit
def kernel(text_input, trainable_weight, frozen_weight, trainable_map,
           frozen_map):
    text3d = text_input.reshape(NW, NCH, CH)
    out = _run(text3d, trainable_weight, frozen_weight, trainable_map,
               frozen_map)
    return out.reshape(text_input.shape[0], text_input.shape[1], D)


# speedup vs baseline: 20.2292x; 14.5560x over previous
import jax
import jax.numpy as jnp
from jax import lax
from jax.experimental import pallas as pl
from jax.experimental.pallas import tpu as pltpu
from jax.experimental.pallas import tpu_sc as plsc

NC, NS, LANES = 2, 16, 16
NW = NC * NS
B = 4096 * 50
D = 64
PER_W = B // NW
CH = 128
NCH = PER_W // CH
R = 5
MAP_AHEAD = 3


def _body(text_hbm, tw_hbm, fw_hbm, tmap_hbm, fmap_hbm, out_hbm,
          tok_v, tidx_v, fidx_v, fw_v, rows_a,
          sem_row, sem_out, sem_map):
    wid = lax.axis_index("s") * NC + lax.axis_index("c")

    pltpu.sync_copy(fw_hbm, fw_v)
    pltpu.sync_copy(text_hbm.at[wid], tok_v)

    def map_copies(c, s):
        return (
            pltpu.make_async_copy(tmap_hbm.at[tok_v.at[c]], tidx_v.at[c],
                                  sem_map.at[s]),
            pltpu.make_async_copy(fmap_hbm.at[tok_v.at[c]], fidx_v.at[c],
                                  sem_map.at[s]),
        )

    def row_copies(c, s):
        return (
            pltpu.make_async_copy(tw_hbm.at[tidx_v.at[c]], rows_a[s],
                                  sem_row.at[s]),
        )

    def out_copy(c, s):
        return pltpu.make_async_copy(
            rows_a[s], out_hbm.at[pl.ds(wid * PER_W + c * CH, CH)],
            sem_out.at[s])

    def issue(copies):
        for cp in copies:
            cp.start()

    def drain(copies):
        for cp in copies:
            cp.wait()

    def compute(c, s):
        @pl.loop(0, CH // LANES)
        def _(g):
            fvec = fidx_v[c, pl.ds(g * LANES, LANES)]
            nfrozen = plsc.all_reduce_population_count(fvec != 0)

            @pl.when(nfrozen[0] > 0)
            def _():
                base = g * LANES
                for k in range(LANES):
                    f = fvec[k]

                    @pl.when(f != 0)
                    def _():
                        for cc in range(D // LANES):
                            sl = pl.ds(cc * LANES, LANES)
                            rows_a[s][base + k, sl] = (
                                rows_a[s][base + k, sl] + fw_v[f, sl])

    for c in range(MAP_AHEAD):
        issue(map_copies(c, c % R))
    drain(map_copies(0, 0))
    issue(row_copies(0, 0))

    @pl.loop(0, NCH, step=R)
    def _(i0):
        for b in range(R):
            i = i0 + b
            drain(row_copies(i, b))
            compute(i, b)
            out_copy(i, b).start()

            j = i + 1
            bj = (b + 1) % R

            @pl.when(j < NCH)
            def _():
                drain(map_copies(j, bj))

                @pl.when(j >= R)
                def _():
                    out_copy(j - R, bj).wait()

                issue(row_copies(j, bj))

            m = i + MAP_AHEAD
            bm = (b + MAP_AHEAD) % R

            @pl.when(m < NCH)
            def _():
                issue(map_copies(m, bm))

    for b in range(R):
        out_copy(NCH - R + b, b).wait()


_run = pl.kernel(
    _body,
    out_type=jax.ShapeDtypeStruct((B, D), jnp.float32),
    mesh=plsc.VectorSubcoreMesh(core_axis_name="c", subcore_axis_name="s"),
    compiler_params=pltpu.CompilerParams(use_tc_tiling_on_sc=False,
                                         needs_layout_passes=False),
    scratch_types=[
        pltpu.VMEM((NCH, CH), jnp.int32),
        pltpu.VMEM((NCH, CH), jnp.int32),
        pltpu.VMEM((NCH, CH), jnp.int32),
        pltpu.VMEM((65, D), jnp.float32),
        [pltpu.VMEM((CH, D), jnp.float32) for _ in range(R)],
        pltpu.SemaphoreType.DMA((R,)),
        pltpu.SemaphoreType.DMA((R,)),
        pltpu.SemaphoreType.DMA((R,)),
    ],
)


@jax.jit
def kernel(text_input, trainable_weight, frozen_weight, trainable_map,
           frozen_map):
    text3d = text_input.reshape(NW, NCH, CH)
    out = _run(text3d, trainable_weight, frozen_weight, trainable_map,
               frozen_map)
    return out.reshape(text_input.shape[0], text_input.shape[1], D)
